# baseline (speedup 1.0000x reference)
import functools

import jax
import jax.numpy as jnp
from jax import lax
from jax.experimental import pallas as pl
from jax.experimental.pallas import tpu as pltpu
from jax.experimental.pallas import tpu_sc as plsc

N_EMBD = 1024
B_TOTAL = 16384 * 200
NC, NS = 2, 16
NW = NC * NS
B_PER_W = B_TOTAL // NW
CHUNK = 40
N_CHUNKS = B_PER_W // CHUNK


def _sc_lookup(styles_flat, table):
    mesh = plsc.VectorSubcoreMesh(core_axis_name="c", subcore_axis_name="s")

    @functools.partial(
        pl.kernel,
        mesh=mesh,
        out_type=jax.ShapeDtypeStruct((B_TOTAL, N_EMBD), jnp.float32),
        scratch_types=[
            pltpu.VMEM((CHUNK,), jnp.int32),
            pltpu.VMEM((CHUNK, N_EMBD), jnp.float32),
            pltpu.SemaphoreType.DMA,
        ],
    )
    def k(idx_hbm, table_hbm, out_hbm, idx_v, rows_v, sem):
        wid = lax.axis_index("s") * NC + lax.axis_index("c")
        base = wid * B_PER_W

        def body(i, carry):
            off = base + i * CHUNK
            pltpu.sync_copy(idx_hbm.at[pl.ds(off, CHUNK)], idx_v)
            pltpu.async_copy(table_hbm.at[idx_v], rows_v, sem).wait()
            pltpu.sync_copy(rows_v, out_hbm.at[pl.ds(off, CHUNK)])
            return carry

        lax.fori_loop(0, N_CHUNKS, body, 0)

    return k(styles_flat, table)


def kernel(styles, table):
    flat = styles.reshape(-1)
    out = _sc_lookup(flat, table)
    return out.reshape(styles.shape + (N_EMBD,))


# speedup vs baseline: 1.0003x; 1.0003x over previous
import functools

import jax
import jax.numpy as jnp
from jax import lax
from jax.experimental import pallas as pl
from jax.experimental.pallas import tpu as pltpu
from jax.experimental.pallas import tpu_sc as plsc

N_EMBD = 1024
B_TOTAL = 16384 * 200
NC, NS = 2, 16
NW = NC * NS
B_PER_W = B_TOTAL // NW
CHUNK = 16
NBUF = 4
N_CHUNKS = B_PER_W // CHUNK
NG = N_CHUNKS // NBUF
STAGE = 8192
CHUNKS_PER_STAGE = STAGE // CHUNK


def _sc_lookup(styles_flat, table):
    mesh = plsc.VectorSubcoreMesh(core_axis_name="c", subcore_axis_name="s")

    @functools.partial(
        pl.kernel,
        mesh=mesh,
        out_type=jax.ShapeDtypeStruct((B_TOTAL, N_EMBD), jnp.float32),
        scratch_types=[
            pltpu.VMEM((STAGE,), jnp.int32),
            pltpu.VMEM((NBUF, CHUNK, N_EMBD), jnp.float32),
            pltpu.SemaphoreType.DMA((NBUF,)),
            pltpu.SemaphoreType.DMA((NBUF,)),
        ],
    )
    def k(idx_hbm, table_hbm, out_hbm, idx_s, rows_v, gsem, ssem):
        wid = lax.axis_index("s") * NC + lax.axis_index("c")
        base = wid * B_PER_W

        pltpu.sync_copy(idx_hbm.at[pl.ds(base, STAGE)], idx_s)
        pltpu.async_copy(
            table_hbm.at[idx_s.at[pl.ds(0, CHUNK)]], rows_v.at[0], gsem.at[0])

        def group(g, carry):
            for b in range(NBUF):
                nb = (b + 1) % NBUF
                i = g * NBUF + b
                pltpu.make_async_copy(
                    table_hbm.at[idx_s.at[pl.ds(0, CHUNK)]],
                    rows_v.at[b], gsem.at[b]).wait()
                @pl.when(lax.rem(i + 1, CHUNKS_PER_STAGE) == 0)
                def _():
                    pltpu.sync_copy(
                        idx_hbm.at[pl.ds(base + (i + 1) * CHUNK, STAGE)],
                        idx_s)
                @pl.when(i + 1 < N_CHUNKS)
                def _():
                    @pl.when(i + 1 >= NBUF)
                    def _():
                        pltpu.make_async_copy(
                            rows_v.at[nb], out_hbm.at[pl.ds(0, CHUNK)],
                            ssem.at[nb]).wait()
                    loc = lax.rem((i + 1) * CHUNK, STAGE)
                    pltpu.async_copy(
                        table_hbm.at[idx_s.at[pl.ds(loc, CHUNK)]],
                        rows_v.at[nb], gsem.at[nb])
                pltpu.async_copy(
                    rows_v.at[b], out_hbm.at[pl.ds(base + i * CHUNK, CHUNK)],
                    ssem.at[b])
            return carry

        lax.fori_loop(0, NG, group, 0)

        for b in range(NBUF):
            pltpu.make_async_copy(
                rows_v.at[b], out_hbm.at[pl.ds(0, CHUNK)], ssem.at[b]).wait()

    return k(styles_flat, table)


def kernel(styles, table):
    flat = styles.reshape(-1)
    out = _sc_lookup(flat, table)
    return out.reshape(styles.shape + (N_EMBD,))


# speedup vs baseline: 4.4617x; 4.4601x over previous
import functools

import jax
import jax.numpy as jnp
from jax import lax
from jax.experimental import pallas as pl
from jax.experimental.pallas import tpu as pltpu
from jax.experimental.pallas import tpu_sc as plsc

N_EMBD = 1024
B_TOTAL = 16384 * 200
NC, NS = 2, 16
NW = NC * NS
B_PER_W = B_TOTAL // NW
CHUNK = 16
NBUF = 4
N_CHUNKS = B_PER_W // CHUNK
NG = N_CHUNKS // NBUF
STAGE = 10240
CHUNKS_PER_STAGE = STAGE // CHUNK


def _sc_lookup(styles_flat, table_rep):
    mesh = plsc.VectorSubcoreMesh(core_axis_name="c", subcore_axis_name="s")

    @functools.partial(
        pl.kernel,
        mesh=mesh,
        out_type=jax.ShapeDtypeStruct((B_TOTAL, N_EMBD), jnp.float32),
        scratch_types=[
            pltpu.VMEM((STAGE,), jnp.int32),
            pltpu.VMEM((NBUF, CHUNK, N_EMBD), jnp.float32),
            pltpu.SemaphoreType.DMA((NBUF,)),
            pltpu.SemaphoreType.DMA((NBUF,)),
        ],
    )
    def k(idx_hbm, table_hbm, out_hbm, idx_s, rows_v, gsem, ssem):
        wid = lax.axis_index("s") * NC + lax.axis_index("c")
        base = wid * B_PER_W
        off_splat = jnp.full((16,), 2 * wid, jnp.int32)

        def stage_fill(start):
            pltpu.sync_copy(idx_hbm.at[pl.ds(start, STAGE)], idx_s)

            def fix(j, c):
                v = idx_s[pl.ds(j * 16, 16)]
                idx_s[pl.ds(j * 16, 16)] = v + off_splat
                return c

            lax.fori_loop(0, STAGE // 16, fix, 0)

        stage_fill(base)
        pltpu.async_copy(
            table_hbm.at[idx_s.at[pl.ds(0, CHUNK)]], rows_v.at[0], gsem.at[0])

        def group(g, carry):
            for b in range(NBUF):
                nb = (b + 1) % NBUF
                i = g * NBUF + b
                pltpu.make_async_copy(
                    table_hbm.at[idx_s.at[pl.ds(0, CHUNK)]],
                    rows_v.at[b], gsem.at[b]).wait()
                @pl.when(lax.rem(i + 1, CHUNKS_PER_STAGE) == 0)
                def _():
                    stage_fill(base + (i + 1) * CHUNK)
                @pl.when(i + 1 < N_CHUNKS)
                def _():
                    @pl.when(i + 1 >= NBUF)
                    def _():
                        pltpu.make_async_copy(
                            rows_v.at[nb], out_hbm.at[pl.ds(0, CHUNK)],
                            ssem.at[nb]).wait()
                    loc = lax.rem((i + 1) * CHUNK, STAGE)
                    pltpu.async_copy(
                        table_hbm.at[idx_s.at[pl.ds(loc, CHUNK)]],
                        rows_v.at[nb], gsem.at[nb])
                pltpu.async_copy(
                    rows_v.at[b], out_hbm.at[pl.ds(base + i * CHUNK, CHUNK)],
                    ssem.at[b])
            return carry

        lax.fori_loop(0, NG, group, 0)

        for b in range(NBUF):
            pltpu.make_async_copy(
                rows_v.at[b], out_hbm.at[pl.ds(0, CHUNK)], ssem.at[b]).wait()

    return k(styles_flat, table_rep)


def kernel(styles, table):
    flat = styles.reshape(-1)
    table_rep = jnp.tile(table, (NW, 1))
    out = _sc_lookup(flat, table_rep)
    return out.reshape(styles.shape + (N_EMBD,))
